# baseline (speedup 1.0000x reference)
import functools

import jax
import jax.numpy as jnp
from jax import lax
from jax.experimental import pallas as pl
from jax.experimental.pallas import tpu as pltpu
from jax.experimental.pallas import tpu_sc as plsc


def _sc_gather(idx, table):
    (BT,) = idx.shape
    _, d = table.shape
    info = plsc.get_sparse_core_info()
    NC, NS = info.num_cores, info.num_subcores
    NW = NC * NS
    per_w = BT // NW
    CHUNK = 640
    NCH = per_w // CHUNK
    mesh = plsc.VectorSubcoreMesh(core_axis_name="c", subcore_axis_name="s")

    @functools.partial(
        pl.kernel,
        mesh=mesh,
        out_type=jax.ShapeDtypeStruct((BT, d), jnp.float32),
        scratch_types=[
            pltpu.VMEM((per_w,), jnp.int32),
            pltpu.VMEM((CHUNK, d), jnp.float32),
            pltpu.SemaphoreType.DMA,
        ],
        compiler_params=pltpu.CompilerParams(use_tc_tiling_on_sc=False),
    )
    def gather_kernel(idx_hbm, table_hbm, out_hbm, idx_v, rows_v, sem):
        wid = lax.axis_index("s") * NC + lax.axis_index("c")
        base = wid * per_w
        pltpu.sync_copy(idx_hbm.at[pl.ds(base, per_w)], idx_v)
        for ch in range(NCH):
            pltpu.async_copy(
                table_hbm.at[idx_v.at[pl.ds(ch * CHUNK, CHUNK)]], rows_v, sem
            ).wait()
            pltpu.sync_copy(rows_v, out_hbm.at[pl.ds(base + ch * CHUNK, CHUNK)])

    return gather_kernel(idx, table)


def _lstm_tc(emb_tm, W, U, b2):
    T, B, d = emb_tm.shape
    u = U.shape[0]
    G = 4 * u
    bB = 512
    grid = (B // bB,)

    def body(emb_ref, w_ref, u_ref, b_ref, out_ref, h_ref, c_ref):
        Wm = w_ref[...]
        Um = u_ref[...]
        bv = b_ref[...]
        h = jnp.zeros((bB, u), jnp.float32)
        c = jnp.zeros((bB, u), jnp.float32)
        for t in range(T):
            z = (jnp.dot(emb_ref[t], Wm, preferred_element_type=jnp.float32)
                 + jnp.dot(h, Um, preferred_element_type=jnp.float32) + bv)
            i = jax.nn.sigmoid(z[:, :u])
            f = jax.nn.sigmoid(z[:, u:2 * u])
            g = jnp.tanh(z[:, 2 * u:3 * u])
            o = jax.nn.sigmoid(z[:, 3 * u:])
            c = f * c + i * g
            h = o * jnp.tanh(c)
            out_ref[t] = h
        h_ref[...] = h
        c_ref[...] = c

    return pl.pallas_call(
        body,
        grid=grid,
        in_specs=[
            pl.BlockSpec((T, bB, d), lambda i: (0, i, 0)),
            pl.BlockSpec((d, G), lambda i: (0, 0)),
            pl.BlockSpec((u, G), lambda i: (0, 0)),
            pl.BlockSpec((1, G), lambda i: (0, 0)),
        ],
        out_specs=[
            pl.BlockSpec((T, bB, u), lambda i: (0, i, 0)),
            pl.BlockSpec((bB, u), lambda i: (i, 0)),
            pl.BlockSpec((bB, u), lambda i: (i, 0)),
        ],
        out_shape=[
            jax.ShapeDtypeStruct((T, B, u), jnp.float32),
            jax.ShapeDtypeStruct((B, u), jnp.float32),
            jax.ShapeDtypeStruct((B, u), jnp.float32),
        ],
        compiler_params=pltpu.CompilerParams(
            dimension_semantics=("arbitrary",),
        ),
    )(emb_tm, W, U, b2)


def kernel(x, E, W, U, b):
    B, T = x.shape
    d = E.shape[1]
    idx = jnp.swapaxes(x, 0, 1).reshape(-1)
    emb_flat = _sc_gather(idx, E)
    emb_tm = emb_flat.reshape(T, B, d)
    out_tm, hT, cT = _lstm_tc(emb_tm, W, U, b.reshape(1, -1))
    return jnp.swapaxes(out_tm, 0, 1), hT, cT


# speedup vs baseline: 1.0180x; 1.0180x over previous
import functools

import jax
import jax.numpy as jnp
from jax import lax
from jax.experimental import pallas as pl
from jax.experimental.pallas import tpu as pltpu
from jax.experimental.pallas import tpu_sc as plsc


def _sc_gather_pairs(idx2, table2):
    (BT,) = idx2.shape
    _, d2 = table2.shape
    info = plsc.get_sparse_core_info()
    NC, NS = info.num_cores, info.num_subcores
    NW = NC * NS
    per_w = BT // NW
    CHUNK = 320
    NCH = per_w // CHUNK
    mesh = plsc.VectorSubcoreMesh(core_axis_name="c", subcore_axis_name="s")

    @functools.partial(
        pl.kernel,
        mesh=mesh,
        out_type=jax.ShapeDtypeStruct((BT, d2), jnp.float32),
        scratch_types=[
            pltpu.VMEM((per_w,), jnp.int32),
            pltpu.VMEM((CHUNK, d2), jnp.float32),
            pltpu.SemaphoreType.DMA,
        ],
        compiler_params=pltpu.CompilerParams(use_tc_tiling_on_sc=True),
    )
    def gather_kernel(idx_hbm, table_hbm, out_hbm, idx_v, rows_v, sem):
        wid = lax.axis_index("s") * NC + lax.axis_index("c")
        base = wid * per_w
        pltpu.sync_copy(idx_hbm.at[pl.ds(base, per_w)], idx_v)
        for ch in range(NCH):
            pltpu.async_copy(
                table_hbm.at[idx_v.at[pl.ds(ch * CHUNK, CHUNK)]], rows_v, sem
            ).wait()
            pltpu.sync_copy(rows_v, out_hbm.at[pl.ds(base + ch * CHUNK, CHUNK)])

    return gather_kernel(idx2, table2)


def _lstm_tc(emb2_tm, xb, W, U, b2):
    T, B, d2 = emb2_tm.shape
    d = d2 // 2
    u = U.shape[0]
    G = 4 * u
    bB = 512
    grid = (B // bB,)

    def body(emb_ref, x_ref, w_ref, u_ref, b_ref, out_ref, h_ref, c_ref):
        Wm = w_ref[...]
        Um = u_ref[...]
        bv = b_ref[...]
        h = jnp.zeros((bB, u), jnp.float32)
        c = jnp.zeros((bB, u), jnp.float32)
        for t in range(T):
            row = emb_ref[t]
            p = x_ref[:, t:t + 1] & 1
            xt = jnp.where(p == 1, row[:, d:], row[:, :d])
            z = (jnp.dot(xt, Wm, preferred_element_type=jnp.float32)
                 + jnp.dot(h, Um, preferred_element_type=jnp.float32) + bv)
            i = jax.nn.sigmoid(z[:, :u])
            f = jax.nn.sigmoid(z[:, u:2 * u])
            g = jnp.tanh(z[:, 2 * u:3 * u])
            o = jax.nn.sigmoid(z[:, 3 * u:])
            c = f * c + i * g
            h = o * jnp.tanh(c)
            out_ref[t] = h
        h_ref[...] = h
        c_ref[...] = c

    return pl.pallas_call(
        body,
        grid=grid,
        in_specs=[
            pl.BlockSpec((T, bB, d2), lambda i: (0, i, 0)),
            pl.BlockSpec((bB, T), lambda i: (i, 0)),
            pl.BlockSpec((d, G), lambda i: (0, 0)),
            pl.BlockSpec((u, G), lambda i: (0, 0)),
            pl.BlockSpec((1, G), lambda i: (0, 0)),
        ],
        out_specs=[
            pl.BlockSpec((T, bB, u), lambda i: (0, i, 0)),
            pl.BlockSpec((bB, u), lambda i: (i, 0)),
            pl.BlockSpec((bB, u), lambda i: (i, 0)),
        ],
        out_shape=[
            jax.ShapeDtypeStruct((T, B, u), jnp.float32),
            jax.ShapeDtypeStruct((B, u), jnp.float32),
            jax.ShapeDtypeStruct((B, u), jnp.float32),
        ],
        compiler_params=pltpu.CompilerParams(
            dimension_semantics=("arbitrary",),
        ),
    )(emb2_tm, xb, W, U, b2)


def kernel(x, E, W, U, b):
    B, T = x.shape
    V, d = E.shape
    table2 = E.reshape(V // 2, 2 * d)
    idx_tm = jnp.swapaxes(x, 0, 1).reshape(-1)
    emb2_flat = _sc_gather_pairs(idx_tm >> 1, table2)
    emb2_tm = emb2_flat.reshape(T, B, 2 * d)
    out_tm, hT, cT = _lstm_tc(emb2_tm, x, W, U, b.reshape(1, -1))
    return jnp.swapaxes(out_tm, 0, 1), hT, cT
